# Initial kernel scaffold; baseline (speedup 1.0000x reference)
import jax
import jax.numpy as jnp
from jax.experimental import pallas as pl


def kernel(x, w1s, w2s):
    raise NotImplementedError("write your pallas kernel here")


# speedup vs baseline: 4.9464x; 4.9464x over previous
import functools

import jax
import jax.numpy as jnp
from jax import lax
from jax.experimental import pallas as pl
from jax.experimental.pallas import tpu as pltpu
from jax.experimental.pallas import tpu_sc as plsc

NIN = 2048
NOUT = 2048
DEPTH = 12
N_LEVELS = DEPTH + 1
N_NODES = 2 ** (DEPTH + 1) - 1
BATCH = 8192

L_DENSE = 9
K_HEAD = 2 ** L_DENSE

BT = 256
N_TILES = BATCH // BT

SC_NC = 2
SC_NS = 16
SC_NW = SC_NC * SC_NS
TOK_PER_W = BATCH // SC_NW
CHUNK = 16
N_CHUNKS = TOK_PER_W // CHUNK
N_DEEP = N_LEVELS - L_DENSE


def _tc_body(x_ref, w1h_ref, w2h_ref, y1_ref, nodes_ref):
    x = x_ref[...]
    s = lax.dot_general(
        x, w1h_ref[...],
        dimension_numbers=(((1,), (1,)), ((), ())),
        preferred_element_type=jnp.float32,
        precision=lax.Precision.HIGHEST,
    )
    cols = lax.broadcasted_iota(jnp.int32, (BT, K_HEAD), 1)
    node = jnp.zeros((BT, 1), jnp.int32)
    m = jnp.zeros((BT, K_HEAD), jnp.float32)
    for _ in range(L_DENSE):
        onehot = (cols == node).astype(jnp.float32)
        score = jnp.sum(s * onehot, axis=1, keepdims=True)
        m = m + onehot * score
        node = node * 2 + 1 + (score > 0.0).astype(jnp.int32)
    y1 = lax.dot_general(
        m, w2h_ref[...],
        dimension_numbers=(((1,), (0,)), ((), ())),
        preferred_element_type=jnp.float32,
        precision=lax.Precision.HIGHEST,
    )
    y1_ref[...] = y1
    nodes_ref[...] = node.reshape(1, 1, BT)


def _tc_stage(x, w1h, w2h):
    return pl.pallas_call(
        _tc_body,
        grid=(N_TILES,),
        in_specs=[
            pl.BlockSpec((BT, NIN), lambda i: (i, 0)),
            pl.BlockSpec((K_HEAD, NIN), lambda i: (0, 0)),
            pl.BlockSpec((K_HEAD, NOUT), lambda i: (0, 0)),
        ],
        out_specs=[
            pl.BlockSpec((BT, NOUT), lambda i: (i, 0)),
            pl.BlockSpec((1, 1, BT), lambda i: (i, 0, 0)),
        ],
        out_shape=[
            jax.ShapeDtypeStruct((BATCH, NOUT), jnp.float32),
            jax.ShapeDtypeStruct((N_TILES, 1, BT), jnp.int32),
        ],
    )(x, w1h, w2h)


def _sc_body(x_hbm, nodes_hbm, w1_hbm, w2_hbm, y1_hbm, out_hbm,
             idx_v, x_v, acc_v, rows_v, sem):
    c = lax.axis_index("c")
    s = lax.axis_index("s")
    wid = s * SC_NC + c
    tile_base = wid * TOK_PER_W
    lane = lax.iota(jnp.int32, 16)

    def chunk_body(ci, carry):
        base = pl.multiple_of(tile_base + ci * CHUNK, 8)
        pltpu.sync_copy(nodes_hbm.at[pl.ds(base, CHUNK)], idx_v)
        pltpu.sync_copy(x_hbm.at[pl.ds(base, CHUNK)], x_v)
        pltpu.sync_copy(y1_hbm.at[pl.ds(base, CHUNK)], acc_v)

        gdn = lax.GatherDimensionNumbers(
            offset_dims=(), collapsed_slice_dims=(0,), start_index_map=(0,))

        def lane_total(v):
            for k in (8, 4, 2, 1):
                idx = jnp.bitwise_xor(lane, k)
                v = v + lax.gather(v, idx[:, None], gdn, (1,),
                                   mode=lax.GatherScatterMode.PROMISE_IN_BOUNDS)
            return v

        def level_body(_, carry2):
            pltpu.async_copy(w1_hbm.at[idx_v], rows_v, sem).wait()
            score_vec = jnp.zeros((16,), jnp.float32)
            for t in range(CHUNK):
                def dot_body(j, acc):
                    off = j * 16
                    return acc + x_v[t, pl.ds(off, 16)] * rows_v[t, pl.ds(off, 16)]
                acc16 = lax.fori_loop(0, NIN // 16, dot_body,
                                      jnp.zeros((16,), jnp.float32),
                                      unroll=8)
                score_vec = jnp.where(lane == t, lane_total(acc16), score_vec)
            idx_vec = idx_v[...]
            pltpu.async_copy(w2_hbm.at[idx_v], rows_v, sem).wait()
            for t in range(CHUNK):
                sc_t = score_vec[t]
                def acc_body(j, carry3):
                    off = j * 16
                    acc_v[t, pl.ds(off, 16)] = (
                        acc_v[t, pl.ds(off, 16)] + sc_t * rows_v[t, pl.ds(off, 16)]
                    )
                    return carry3
                lax.fori_loop(0, NOUT // 16, acc_body, 0, unroll=8)
            one = jnp.full((16,), 1, jnp.int32)
            zero = jnp.full((16,), 0, jnp.int32)
            idx_v[...] = idx_vec * 2 + 1 + jnp.where(score_vec > 0.0, one, zero)
            return carry2

        lax.fori_loop(0, N_DEEP, level_body, 0)
        pltpu.sync_copy(acc_v, out_hbm.at[pl.ds(base, CHUNK)])
        return carry

    lax.fori_loop(0, N_CHUNKS, chunk_body, 0)


def _sc_stage(x, nodes, w1s, w2s, y1):
    mesh = plsc.VectorSubcoreMesh(core_axis_name="c", subcore_axis_name="s",
                                  num_cores=SC_NC, num_subcores=SC_NS)
    f = pl.kernel(
        _sc_body,
        out_type=jax.ShapeDtypeStruct((BATCH, NOUT), jnp.float32),
        mesh=mesh,
        scratch_types=[
            pltpu.VMEM((CHUNK,), jnp.int32),
            pltpu.VMEM((CHUNK, NIN), jnp.float32),
            pltpu.VMEM((CHUNK, NOUT), jnp.float32),
            pltpu.VMEM((CHUNK, NIN), jnp.float32),
            pltpu.SemaphoreType.DMA,
        ],
    )
    return f(x, nodes, w1s, w2s, y1)


def kernel(x, w1s, w2s):
    w1h = w1s[:K_HEAD]
    w2h = w2s[:K_HEAD]
    y1, nodes = _tc_stage(x, w1h, w2h)
    nodes_flat = nodes.reshape(BATCH)
    return _sc_stage(x, nodes_flat, w1s, w2s, y1)


# speedup vs baseline: 5.3427x; 1.0801x over previous
import functools

import jax
import jax.numpy as jnp
from jax import lax
from jax.experimental import pallas as pl
from jax.experimental.pallas import tpu as pltpu
from jax.experimental.pallas import tpu_sc as plsc

NIN = 2048
NOUT = 2048
DEPTH = 12
N_LEVELS = DEPTH + 1
N_NODES = 2 ** (DEPTH + 1) - 1
BATCH = 8192

L_DENSE = 9
K_HEAD = 2 ** L_DENSE

BT = 256
N_TILES = BATCH // BT

SC_NC = 2
SC_NS = 16
SC_NW = SC_NC * SC_NS
TOK_PER_W = BATCH // SC_NW
CHUNK = 16
N_CHUNKS = TOK_PER_W // CHUNK
N_DEEP = N_LEVELS - L_DENSE


def _tc_body(x_ref, w1h_ref, w2h_ref, y1_ref, nodes_ref):
    x = x_ref[...]
    s = lax.dot_general(
        x, w1h_ref[...],
        dimension_numbers=(((1,), (1,)), ((), ())),
        preferred_element_type=jnp.float32,
        precision=lax.Precision.HIGHEST,
    )
    cols = lax.broadcasted_iota(jnp.int32, (BT, K_HEAD), 1)
    node = jnp.zeros((BT, 1), jnp.int32)
    m = jnp.zeros((BT, K_HEAD), jnp.float32)
    for _ in range(L_DENSE):
        onehot = (cols == node).astype(jnp.float32)
        score = jnp.sum(s * onehot, axis=1, keepdims=True)
        m = m + onehot * score
        node = node * 2 + 1 + (score > 0.0).astype(jnp.int32)
    y1 = lax.dot_general(
        m, w2h_ref[...],
        dimension_numbers=(((1,), (0,)), ((), ())),
        preferred_element_type=jnp.float32,
        precision=lax.Precision.DEFAULT,
    )
    y1_ref[...] = y1
    nodes_ref[...] = node.reshape(1, 1, BT)


def _tc_stage(x, w1h, w2h):
    return pl.pallas_call(
        _tc_body,
        grid=(N_TILES,),
        in_specs=[
            pl.BlockSpec((BT, NIN), lambda i: (i, 0)),
            pl.BlockSpec((K_HEAD, NIN), lambda i: (0, 0)),
            pl.BlockSpec((K_HEAD, NOUT), lambda i: (0, 0)),
        ],
        out_specs=[
            pl.BlockSpec((BT, NOUT), lambda i: (i, 0)),
            pl.BlockSpec((1, 1, BT), lambda i: (i, 0, 0)),
        ],
        out_shape=[
            jax.ShapeDtypeStruct((BATCH, NOUT), jnp.float32),
            jax.ShapeDtypeStruct((N_TILES, 1, BT), jnp.int32),
        ],
    )(x, w1h, w2h)


def _sc_body(x_hbm, nodes_hbm, w1_hbm, w2_hbm, y1_hbm, out_hbm,
             idx_v, x_v, acc_v, rows_v, sem):
    c = lax.axis_index("c")
    s = lax.axis_index("s")
    wid = s * SC_NC + c
    tile_base = wid * TOK_PER_W
    lane = lax.iota(jnp.int32, 16)

    def chunk_body(ci, carry):
        base = pl.multiple_of(tile_base + ci * CHUNK, 8)
        pltpu.sync_copy(nodes_hbm.at[pl.ds(base, CHUNK)], idx_v)
        pltpu.sync_copy(x_hbm.at[pl.ds(base, CHUNK)], x_v)
        pltpu.sync_copy(y1_hbm.at[pl.ds(base, CHUNK)], acc_v)

        gdn = lax.GatherDimensionNumbers(
            offset_dims=(), collapsed_slice_dims=(0,), start_index_map=(0,))

        def lane_total(v):
            for k in (8, 4, 2, 1):
                idx = jnp.bitwise_xor(lane, k)
                v = v + lax.gather(v, idx[:, None], gdn, (1,),
                                   mode=lax.GatherScatterMode.PROMISE_IN_BOUNDS)
            return v

        def level_body(_, carry2):
            pltpu.async_copy(w1_hbm.at[idx_v], rows_v, sem).wait()
            score_vec = jnp.zeros((16,), jnp.float32)
            for t in range(CHUNK):
                def dot_body(j, acc):
                    off = j * 16
                    return acc + x_v[t, pl.ds(off, 16)] * rows_v[t, pl.ds(off, 16)]
                acc16 = lax.fori_loop(0, NIN // 16, dot_body,
                                      jnp.zeros((16,), jnp.float32),
                                      unroll=8)
                score_vec = jnp.where(lane == t, lane_total(acc16), score_vec)
            idx_vec = idx_v[...]
            pltpu.async_copy(w2_hbm.at[idx_v], rows_v, sem).wait()
            for t in range(CHUNK):
                sc_t = score_vec[t]
                def acc_body(j, carry3):
                    off = j * 16
                    acc_v[t, pl.ds(off, 16)] = (
                        acc_v[t, pl.ds(off, 16)] + sc_t * rows_v[t, pl.ds(off, 16)]
                    )
                    return carry3
                lax.fori_loop(0, NOUT // 16, acc_body, 0, unroll=8)
            one = jnp.full((16,), 1, jnp.int32)
            zero = jnp.full((16,), 0, jnp.int32)
            idx_v[...] = idx_vec * 2 + 1 + jnp.where(score_vec > 0.0, one, zero)
            return carry2

        lax.fori_loop(0, N_DEEP, level_body, 0)
        pltpu.sync_copy(acc_v, out_hbm.at[pl.ds(base, CHUNK)])
        return carry

    lax.fori_loop(0, N_CHUNKS, chunk_body, 0)


def _sc_stage(x, nodes, w1s, w2s, y1):
    mesh = plsc.VectorSubcoreMesh(core_axis_name="c", subcore_axis_name="s",
                                  num_cores=SC_NC, num_subcores=SC_NS)
    f = pl.kernel(
        _sc_body,
        out_type=jax.ShapeDtypeStruct((BATCH, NOUT), jnp.float32),
        mesh=mesh,
        scratch_types=[
            pltpu.VMEM((CHUNK,), jnp.int32),
            pltpu.VMEM((CHUNK, NIN), jnp.float32),
            pltpu.VMEM((CHUNK, NOUT), jnp.float32),
            pltpu.VMEM((CHUNK, NIN), jnp.float32),
            pltpu.SemaphoreType.DMA,
        ],
    )
    return f(x, nodes, w1s, w2s, y1)


def kernel(x, w1s, w2s):
    w1h = w1s[:K_HEAD]
    w2h = w2s[:K_HEAD]
    y1, nodes = _tc_stage(x, w1h, w2h)
    nodes_flat = nodes.reshape(BATCH)
    return _sc_stage(x, nodes_flat, w1s, w2s, y1)
